# baseline (speedup 1.0000x reference)
import functools

import jax
import jax.numpy as jnp
from jax import lax
from jax.experimental import pallas as pl
from jax.experimental.pallas import tpu as pltpu
from jax.experimental.pallas import tpu_sc as plsc

N_NODES = 10000
N_HEDGES = 10000
N_INC = 320000
F_IN = 128
F_OUT = 128
N_HEADS = 4
LN_EPS = 1e-5

D_SIDE = 16
K_CHUNK = 80


def _make_sc_pass(side_gather: bool, g_row: int, s_row: int):
    info = plsc.get_sparse_core_info()
    nc, ns, nl = info.num_cores, info.num_subcores, info.num_lanes
    nw = nc * ns
    epw = N_INC // nw
    chunks = epw // K_CHUNK
    assert N_INC == nw * K_CHUNK * chunks
    assert chunks % 3 == 2
    rows_per_tile = N_HEDGES // ns
    zrows = 25
    assert rows_per_tile % zrows == 0

    mesh = plsc.VectorSubcoreMesh(core_axis_name="c", subcore_axis_name="s")

    scratch = [
        pltpu.VMEM((3, 2, K_CHUNK), jnp.int32),
        pltpu.VMEM((K_CHUNK, F_IN), jnp.float32),
        pltpu.VMEM((K_CHUNK, F_IN), jnp.float32),
        pltpu.VMEM((K_CHUNK, F_IN), jnp.float32),
        pltpu.VMEM((K_CHUNK, D_SIDE), jnp.float32),
        pltpu.VMEM((K_CHUNK, D_SIDE), jnp.float32),
        pltpu.VMEM((K_CHUNK, D_SIDE), jnp.float32),
        pltpu.VMEM((zrows, F_IN), jnp.float32),
        pltpu.VMEM((5 * zrows, D_SIDE), jnp.float32),
        pltpu.VMEM_SHARED((N_HEDGES, F_IN), jnp.float32),
        pltpu.VMEM_SHARED((N_HEDGES, D_SIDE), jnp.float32),
    ] + [pltpu.SemaphoreType.DMA] * 12

    out_type = (
        jax.ShapeDtypeStruct((nc, N_HEDGES, F_IN), jnp.float32),
        jax.ShapeDtypeStruct((nc, N_HEDGES, D_SIDE), jnp.float32),
    )

    def body(table_hbm, idx_hbm, side_hbm, out_hbm, outs_hbm,
             idxb, rows0, rows1, rows2, srow0, srow1, srow2, zbuf_v, zbufs_v,
             acc_sh, accs_sh,
             g0, g1, g2, s0, s1, s2, ig0, ig1, ig2, is0, is1, is2):
        cid = lax.axis_index("c")
        sid = lax.axis_index("s")
        wid = cid * ns + sid
        rows = (rows0, rows1, rows2)
        srow = (srow0, srow1, srow2)
        gsem = (g0, g1, g2)
        ssem = (s0, s1, s2)
        igsem = (ig0, ig1, ig2)
        issem = (is0, is1, is2)

        def zrow(i, carry):
            def zcol(j, c2):
                zbuf_v[i, pl.ds(j * nl, nl)] = jnp.zeros((nl,), jnp.float32)
                return c2
            return lax.fori_loop(0, F_IN // nl, zcol, carry)
        lax.fori_loop(0, zrows, zrow, 0)

        def zrow_s(i, carry):
            zbufs_v[i, pl.ds(0, nl)] = jnp.zeros((nl,), jnp.float32)
            return carry
        lax.fori_loop(0, 5 * zrows, zrow_s, 0)
        base_row = sid * rows_per_tile
        for t in range(rows_per_tile // zrows):
            pltpu.sync_copy(zbuf_v, acc_sh.at[pl.ds(base_row + t * zrows, zrows)])
        for t in range(rows_per_tile // (5 * zrows)):
            pltpu.sync_copy(
                zbufs_v,
                accs_sh.at[pl.ds(base_row + t * 5 * zrows, 5 * zrows)])
        if not side_gather:
            def onerow(i, carry):
                srow0[i, pl.ds(0, nl)] = jnp.ones((nl,), jnp.float32)
                return carry
            lax.fori_loop(0, K_CHUNK, onerow, 0)
        plsc.subcore_barrier()

        def idx_fetch(slot, j):
            pltpu.async_copy(idx_hbm.at[wid, j], idxb.at[slot], igsem[slot])

        def idx_wait(slot, j):
            pltpu.make_async_copy(
                idx_hbm.at[wid, j], idxb.at[slot], igsem[slot]).wait()

        def gather_start(slot):
            pltpu.async_copy(
                table_hbm.at[idxb.at[slot, 0]], rows[slot], gsem[slot])
            if side_gather:
                pltpu.async_copy(
                    side_hbm.at[idxb.at[slot, 0]], srow[slot], ssem[slot])

        def gather_wait(slot):
            pltpu.make_async_copy(
                table_hbm.at[idxb.at[slot, 0]], rows[slot], gsem[slot]).wait()
            if side_gather:
                pltpu.make_async_copy(
                    side_hbm.at[idxb.at[slot, 0]], srow[slot], ssem[slot]).wait()

        def scatter(slot):
            pltpu.sync_copy(rows[slot], acc_sh.at[idxb.at[slot, 1]], add=True)
            sb = srow[slot] if side_gather else srow0
            pltpu.sync_copy(sb, accs_sh.at[idxb.at[slot, 1]], add=True)

        idx_fetch(0, 0)
        idx_wait(0, 0)
        gather_start(0)
        idx_fetch(1, 1)
        idx_wait(1, 1)
        gather_start(1)
        idx_fetch(2, 2)

        def loop_body(t, carry):
            for b in range(3):
                c = 3 * t + b
                b2 = (b + 2) % 3
                gather_wait(b)
                idx_wait(b2, c + 2)
                gather_start(b2)
                scatter(b)
                @pl.when(c + 3 < chunks)
                def _():
                    idx_fetch(b, c + 3)
            return carry
        lax.fori_loop(0, chunks // 3, loop_body, 0)

        for b in range(2):
            gather_wait(b)
            scatter(b)

        plsc.subcore_barrier()
        pltpu.sync_copy(acc_sh.at[pl.ds(base_row, rows_per_tile)],
                        out_hbm.at[cid, pl.ds(base_row, rows_per_tile)])
        pltpu.sync_copy(accs_sh.at[pl.ds(base_row, rows_per_tile)],
                        outs_hbm.at[cid, pl.ds(base_row, rows_per_tile)])

    if side_gather:
        fn = body
    else:
        def fn(table_hbm, idx_hbm, out_hbm, outs_hbm,
               idxb, rows0, rows1, rows2, srow0, srow1, srow2, zbuf_v, zbufs_v,
               acc_sh, accs_sh,
               g0, g1, g2, s0, s1, s2, ig0, ig1, ig2, is0, is1, is2):
            return body(table_hbm, idx_hbm, table_hbm, out_hbm, outs_hbm,
                        idxb, rows0, rows1, rows2, srow0, srow1, srow2,
                        zbuf_v, zbufs_v, acc_sh, accs_sh,
                        g0, g1, g2, s0, s1, s2, ig0, ig1, ig2, is0, is1, is2)

    return functools.partial(
        pl.kernel,
        mesh=mesh,
        compiler_params=pltpu.CompilerParams(use_tc_tiling_on_sc=False),
        out_type=out_type,
        scratch_types=scratch,
    )(fn)


def _mid_body(yp_ref, deg_ref, o_ref):
    ysum = yp_ref[0] + yp_ref[1]
    deg = deg_ref[...]
    binv = jnp.where(deg > 0, 1.0 / deg, 0.0)
    o_ref[...] = ysum * binv


def _final_body(ap_ref, deg_ref, w_ref, b_ref, g_ref, be_ref, o_ref):
    asum = ap_ref[0] + ap_ref[1]
    deg = deg_ref[...]
    dinv = jnp.where(deg > 0, 1.0 / deg, 0.0)
    a = asum * dinv
    h = jnp.dot(a, w_ref[...], preferred_element_type=jnp.float32,
                precision=lax.Precision.HIGHEST)
    h = jnp.maximum(h + b_ref[...], 0.0)
    mu = jnp.mean(h, axis=1, keepdims=True)
    var = jnp.mean((h - mu) ** 2, axis=1, keepdims=True)
    o_ref[...] = (h - mu) * lax.rsqrt(var + LN_EPS) * g_ref[...] + be_ref[...]


def kernel(x, edge, edge_weight, W0, b0, W1, b1, W2, b2, W3, b3, gamma, beta):
    sc_pass1 = _make_sc_pass(side_gather=False, g_row=0, s_row=1)
    sc_pass2 = _make_sc_pass(side_gather=True, g_row=1, s_row=0)

    info = plsc.get_sparse_core_info()
    nw = info.num_cores * info.num_subcores
    chunks = N_INC // (nw * K_CHUNK)
    e4 = edge.reshape(2, nw, chunks, K_CHUNK)
    idx1 = jnp.stack([e4[0], e4[1]], axis=2)
    idx2 = jnp.stack([e4[1], e4[0]], axis=2)

    ypart, cpart = sc_pass1(x, idx1)
    deg_e = (cpart[0, :, 0] + cpart[1, :, 0]).reshape(N_HEDGES, 1)

    br = 2000
    hf = N_HEADS * F_OUT
    yscaled = pl.pallas_call(
        _mid_body,
        grid=(N_HEDGES // br,),
        in_specs=[
            pl.BlockSpec((2, br, F_IN), lambda i: (0, i, 0)),
            pl.BlockSpec((br, 1), lambda i: (i, 0)),
        ],
        out_specs=pl.BlockSpec((br, F_IN), lambda i: (i, 0)),
        out_shape=jax.ShapeDtypeStruct((N_HEDGES, F_IN), jnp.float32),
    )(ypart, deg_e)

    ew16 = jnp.broadcast_to(edge_weight.reshape(N_HEDGES, 1),
                            (N_HEDGES, D_SIDE))
    apart, wpart = sc_pass2(yscaled, idx2, ew16)
    deg_v = (wpart[0, :, 0] + wpart[1, :, 0]).reshape(N_NODES, 1)

    wcat = jnp.concatenate([W0, W1, W2, W3], axis=1)
    bcat = jnp.concatenate([b0, b1, b2, b3]).reshape(1, hf)
    out = pl.pallas_call(
        _final_body,
        grid=(N_NODES // br,),
        in_specs=[
            pl.BlockSpec((2, br, F_IN), lambda i: (0, i, 0)),
            pl.BlockSpec((br, 1), lambda i: (i, 0)),
            pl.BlockSpec((F_IN, hf), lambda i: (0, 0)),
            pl.BlockSpec((1, hf), lambda i: (0, 0)),
            pl.BlockSpec((1, hf), lambda i: (0, 0)),
            pl.BlockSpec((1, hf), lambda i: (0, 0)),
        ],
        out_specs=pl.BlockSpec((br, hf), lambda i: (i, 0)),
        out_shape=jax.ShapeDtypeStruct((N_NODES, hf), jnp.float32),
    )(apart, deg_v, wcat, bcat, gamma.reshape(1, hf), beta.reshape(1, hf))

    return out


# speedup vs baseline: 1.1290x; 1.1290x over previous
import functools

import jax
import jax.numpy as jnp
from jax import lax
from jax.experimental import pallas as pl
from jax.experimental.pallas import tpu as pltpu
from jax.experimental.pallas import tpu_sc as plsc

N_NODES = 10000
N_HEDGES = 10000
N_INC = 320000
F_IN = 128
F_OUT = 128
N_HEADS = 4
LN_EPS = 1e-5

D_SIDE = 16
K_CHUNK = 80


def _make_sc_pass(side_gather: bool, g_row: int, s_row: int):
    info = plsc.get_sparse_core_info()
    nc, ns, nl = info.num_cores, info.num_subcores, info.num_lanes
    nw = nc * ns
    epw = N_INC // nw
    chunks = epw // K_CHUNK
    assert N_INC == nw * K_CHUNK * chunks
    assert chunks % 3 == 2
    rows_per_tile = N_HEDGES // ns
    zrows = 25
    assert rows_per_tile % zrows == 0

    mesh = plsc.VectorSubcoreMesh(core_axis_name="c", subcore_axis_name="s")

    scratch = [
        pltpu.VMEM((3, 2, K_CHUNK), jnp.int32),
        pltpu.VMEM((K_CHUNK, F_IN), jnp.float32),
        pltpu.VMEM((K_CHUNK, F_IN), jnp.float32),
        pltpu.VMEM((K_CHUNK, F_IN), jnp.float32),
        pltpu.VMEM((K_CHUNK, D_SIDE), jnp.float32),
        pltpu.VMEM((K_CHUNK, D_SIDE), jnp.float32),
        pltpu.VMEM((K_CHUNK, D_SIDE), jnp.float32),
        pltpu.VMEM((zrows, F_IN), jnp.float32),
        pltpu.VMEM((5 * zrows, D_SIDE), jnp.float32),
        pltpu.VMEM_SHARED((N_HEDGES, F_IN), jnp.float32),
        pltpu.VMEM_SHARED((N_HEDGES, D_SIDE), jnp.float32),
    ] + [pltpu.SemaphoreType.DMA] * 12

    out_type = (
        jax.ShapeDtypeStruct((nc, N_HEDGES, F_IN), jnp.float32),
        jax.ShapeDtypeStruct((nc, N_HEDGES, D_SIDE), jnp.float32),
    )

    def body(table_hbm, idx_hbm, side_hbm, out_hbm, outs_hbm,
             idxb, rows0, rows1, rows2, srow0, srow1, srow2, zbuf_v, zbufs_v,
             acc_sh, accs_sh,
             g0, g1, g2, s0, s1, s2, ig0, ig1, ig2, is0, is1, is2):
        cid = lax.axis_index("c")
        sid = lax.axis_index("s")
        wid = cid * ns + sid
        rows = (rows0, rows1, rows2)
        srow = (srow0, srow1, srow2)
        gsem = (g0, g1, g2)
        ssem = (s0, s1, s2)
        igsem = (ig0, ig1, ig2)
        issem = (is0, is1, is2)

        def zrow(i, carry):
            def zcol(j, c2):
                zbuf_v[i, pl.ds(j * nl, nl)] = jnp.zeros((nl,), jnp.float32)
                return c2
            return lax.fori_loop(0, F_IN // nl, zcol, carry)
        lax.fori_loop(0, zrows, zrow, 0)

        def zrow_s(i, carry):
            zbufs_v[i, pl.ds(0, nl)] = jnp.zeros((nl,), jnp.float32)
            return carry
        lax.fori_loop(0, 5 * zrows, zrow_s, 0)
        base_row = sid * rows_per_tile
        for t in range(rows_per_tile // zrows):
            pltpu.sync_copy(zbuf_v, acc_sh.at[pl.ds(base_row + t * zrows, zrows)])
        for t in range(rows_per_tile // (5 * zrows)):
            pltpu.sync_copy(
                zbufs_v,
                accs_sh.at[pl.ds(base_row + t * 5 * zrows, 5 * zrows)])
        if not side_gather:
            def onerow(i, carry):
                srow0[i, pl.ds(0, nl)] = jnp.ones((nl,), jnp.float32)
                return carry
            lax.fori_loop(0, K_CHUNK, onerow, 0)
        plsc.subcore_barrier()

        def idx_fetch(slot, j):
            pltpu.async_copy(idx_hbm.at[wid, j], idxb.at[slot], igsem[slot])

        def idx_wait(slot, j):
            pltpu.make_async_copy(
                idx_hbm.at[wid, j], idxb.at[slot], igsem[slot]).wait()

        def gather_start(slot):
            pltpu.async_copy(
                table_hbm.at[idxb.at[slot, 0]], rows[slot], gsem[slot])
            if side_gather:
                pltpu.async_copy(
                    side_hbm.at[idxb.at[slot, 0]], srow[slot], ssem[slot])

        def gather_wait(slot):
            pltpu.make_async_copy(
                table_hbm.at[idxb.at[slot, 0]], rows[slot], gsem[slot]).wait()
            if side_gather:
                pltpu.make_async_copy(
                    side_hbm.at[idxb.at[slot, 0]], srow[slot], ssem[slot]).wait()

        def scatter(slot):
            sb = srow[slot] if side_gather else srow0
            pltpu.async_copy(rows[slot], acc_sh.at[idxb.at[slot, 1]],
                             issem[slot], add=True)
            pltpu.async_copy(sb, accs_sh.at[idxb.at[slot, 1]],
                             issem[slot], add=True)
            pltpu.make_async_copy(rows[slot], acc_sh.at[idxb.at[slot, 1]],
                                  issem[slot]).wait()
            pltpu.make_async_copy(sb, accs_sh.at[idxb.at[slot, 1]],
                                  issem[slot]).wait()

        idx_fetch(0, 0)
        idx_wait(0, 0)
        gather_start(0)
        idx_fetch(1, 1)
        idx_wait(1, 1)
        gather_start(1)
        idx_fetch(2, 2)

        def loop_body(t, carry):
            for b in range(3):
                c = 3 * t + b
                b2 = (b + 2) % 3
                gather_wait(b)
                scatter(b)
                @pl.when(c + 3 < chunks)
                def _():
                    idx_fetch(b, c + 3)
                idx_wait(b2, c + 2)
                gather_start(b2)
            return carry
        lax.fori_loop(0, chunks // 3, loop_body, 0)

        for b in range(2):
            gather_wait(b)
            scatter(b)

        plsc.subcore_barrier()
        pltpu.sync_copy(acc_sh.at[pl.ds(base_row, rows_per_tile)],
                        out_hbm.at[cid, pl.ds(base_row, rows_per_tile)])
        pltpu.sync_copy(accs_sh.at[pl.ds(base_row, rows_per_tile)],
                        outs_hbm.at[cid, pl.ds(base_row, rows_per_tile)])

    if side_gather:
        fn = body
    else:
        def fn(table_hbm, idx_hbm, out_hbm, outs_hbm,
               idxb, rows0, rows1, rows2, srow0, srow1, srow2, zbuf_v, zbufs_v,
               acc_sh, accs_sh,
               g0, g1, g2, s0, s1, s2, ig0, ig1, ig2, is0, is1, is2):
            return body(table_hbm, idx_hbm, table_hbm, out_hbm, outs_hbm,
                        idxb, rows0, rows1, rows2, srow0, srow1, srow2,
                        zbuf_v, zbufs_v, acc_sh, accs_sh,
                        g0, g1, g2, s0, s1, s2, ig0, ig1, ig2, is0, is1, is2)

    return functools.partial(
        pl.kernel,
        mesh=mesh,
        compiler_params=pltpu.CompilerParams(use_tc_tiling_on_sc=False),
        out_type=out_type,
        scratch_types=scratch,
    )(fn)


def _mid_body(yp_ref, deg_ref, o_ref):
    ysum = yp_ref[0] + yp_ref[1]
    deg = deg_ref[...]
    binv = jnp.where(deg > 0, 1.0 / deg, 0.0)
    o_ref[...] = ysum * binv


def _final_body(ap_ref, deg_ref, w_ref, b_ref, g_ref, be_ref, o_ref):
    asum = ap_ref[0] + ap_ref[1]
    deg = deg_ref[...]
    dinv = jnp.where(deg > 0, 1.0 / deg, 0.0)
    a = asum * dinv
    h = jnp.dot(a, w_ref[...], preferred_element_type=jnp.float32,
                precision=lax.Precision.HIGHEST)
    h = jnp.maximum(h + b_ref[...], 0.0)
    mu = jnp.mean(h, axis=1, keepdims=True)
    var = jnp.mean((h - mu) ** 2, axis=1, keepdims=True)
    o_ref[...] = (h - mu) * lax.rsqrt(var + LN_EPS) * g_ref[...] + be_ref[...]


def kernel(x, edge, edge_weight, W0, b0, W1, b1, W2, b2, W3, b3, gamma, beta):
    sc_pass1 = _make_sc_pass(side_gather=False, g_row=0, s_row=1)
    sc_pass2 = _make_sc_pass(side_gather=True, g_row=1, s_row=0)

    info = plsc.get_sparse_core_info()
    nw = info.num_cores * info.num_subcores
    chunks = N_INC // (nw * K_CHUNK)
    e4 = edge.reshape(2, nw, chunks, K_CHUNK)
    idx1 = jnp.stack([e4[0], e4[1]], axis=2)
    idx2 = jnp.stack([e4[1], e4[0]], axis=2)

    ypart, cpart = sc_pass1(x, idx1)
    deg_e = (cpart[0, :, 0] + cpart[1, :, 0]).reshape(N_HEDGES, 1)

    br = 2000
    hf = N_HEADS * F_OUT
    yscaled = pl.pallas_call(
        _mid_body,
        grid=(N_HEDGES // br,),
        in_specs=[
            pl.BlockSpec((2, br, F_IN), lambda i: (0, i, 0)),
            pl.BlockSpec((br, 1), lambda i: (i, 0)),
        ],
        out_specs=pl.BlockSpec((br, F_IN), lambda i: (i, 0)),
        out_shape=jax.ShapeDtypeStruct((N_HEDGES, F_IN), jnp.float32),
    )(ypart, deg_e)

    ew16 = jnp.broadcast_to(edge_weight.reshape(N_HEDGES, 1),
                            (N_HEDGES, D_SIDE))
    apart, wpart = sc_pass2(yscaled, idx2, ew16)
    deg_v = (wpart[0, :, 0] + wpart[1, :, 0]).reshape(N_NODES, 1)

    wcat = jnp.concatenate([W0, W1, W2, W3], axis=1)
    bcat = jnp.concatenate([b0, b1, b2, b3]).reshape(1, hf)
    out = pl.pallas_call(
        _final_body,
        grid=(N_NODES // br,),
        in_specs=[
            pl.BlockSpec((2, br, F_IN), lambda i: (0, i, 0)),
            pl.BlockSpec((br, 1), lambda i: (i, 0)),
            pl.BlockSpec((F_IN, hf), lambda i: (0, 0)),
            pl.BlockSpec((1, hf), lambda i: (0, 0)),
            pl.BlockSpec((1, hf), lambda i: (0, 0)),
            pl.BlockSpec((1, hf), lambda i: (0, 0)),
        ],
        out_specs=pl.BlockSpec((br, hf), lambda i: (i, 0)),
        out_shape=jax.ShapeDtypeStruct((N_NODES, hf), jnp.float32),
    )(apart, deg_v, wcat, bcat, gamma.reshape(1, hf), beta.reshape(1, hf))

    return out


# speedup vs baseline: 1.1625x; 1.0297x over previous
import functools

import jax
import jax.numpy as jnp
from jax import lax
from jax.experimental import pallas as pl
from jax.experimental.pallas import tpu as pltpu
from jax.experimental.pallas import tpu_sc as plsc

N_NODES = 10000
N_HEDGES = 10000
N_INC = 320000
F_IN = 128
F_OUT = 128
N_HEADS = 4
LN_EPS = 1e-5

D_SIDE = 16
K_CHUNK = 80


def _make_sc_pass(side_gather: bool, g_row: int, s_row: int):
    info = plsc.get_sparse_core_info()
    nc, ns, nl = info.num_cores, info.num_subcores, info.num_lanes
    nw = nc * ns
    epw = N_INC // nw
    chunks = epw // K_CHUNK
    assert N_INC == nw * K_CHUNK * chunks
    assert chunks % 3 == 2
    rows_per_tile = N_HEDGES // ns
    zrows = 25
    assert rows_per_tile % zrows == 0

    mesh = plsc.VectorSubcoreMesh(core_axis_name="c", subcore_axis_name="s")

    scratch = [
        pltpu.VMEM((3, 2, K_CHUNK), jnp.int32),
        pltpu.VMEM((K_CHUNK, F_IN), jnp.float32),
        pltpu.VMEM((K_CHUNK, F_IN), jnp.float32),
        pltpu.VMEM((K_CHUNK, F_IN), jnp.float32),
        pltpu.VMEM((K_CHUNK, D_SIDE), jnp.float32),
        pltpu.VMEM((K_CHUNK, D_SIDE), jnp.float32),
        pltpu.VMEM((K_CHUNK, D_SIDE), jnp.float32),
        pltpu.VMEM((zrows, F_IN), jnp.float32),
        pltpu.VMEM((5 * zrows, D_SIDE), jnp.float32),
        pltpu.VMEM_SHARED((N_HEDGES, F_IN), jnp.float32),
        pltpu.VMEM_SHARED((N_HEDGES, D_SIDE), jnp.float32),
    ] + [pltpu.SemaphoreType.DMA] * 12

    out_type = (
        jax.ShapeDtypeStruct((nc, N_HEDGES, F_IN), jnp.float32),
        jax.ShapeDtypeStruct((nc, N_HEDGES, D_SIDE), jnp.float32),
    )

    def body(table_hbm, idx_hbm, side_hbm, out_hbm, outs_hbm,
             idxb, rows0, rows1, rows2, srow0, srow1, srow2, zbuf_v, zbufs_v,
             acc_sh, accs_sh,
             g0, g1, g2, s0, s1, s2, ig0, ig1, ig2, is0, is1, is2):
        cid = lax.axis_index("c")
        sid = lax.axis_index("s")
        wid = cid * ns + sid
        rows = (rows0, rows1, rows2)
        srow = (srow0, srow1, srow2)
        gsem = (g0, g1, g2)
        ssem = (s0, s1, s2)
        igsem = (ig0, ig1, ig2)
        issem = (is0, is1, is2)

        def zrow(i, carry):
            def zcol(j, c2):
                zbuf_v[i, pl.ds(j * nl, nl)] = jnp.zeros((nl,), jnp.float32)
                return c2
            return lax.fori_loop(0, F_IN // nl, zcol, carry)
        lax.fori_loop(0, zrows, zrow, 0)

        def zrow_s(i, carry):
            zbufs_v[i, pl.ds(0, nl)] = jnp.zeros((nl,), jnp.float32)
            return carry
        lax.fori_loop(0, 5 * zrows, zrow_s, 0)
        base_row = sid * rows_per_tile
        for t in range(rows_per_tile // zrows):
            pltpu.sync_copy(zbuf_v, acc_sh.at[pl.ds(base_row + t * zrows, zrows)])
        for t in range(rows_per_tile // (5 * zrows)):
            pltpu.sync_copy(
                zbufs_v,
                accs_sh.at[pl.ds(base_row + t * 5 * zrows, 5 * zrows)])
        if not side_gather:
            def onerow(i, carry):
                srow0[i, pl.ds(0, nl)] = jnp.ones((nl,), jnp.float32)
                return carry
            lax.fori_loop(0, K_CHUNK, onerow, 0)
        plsc.subcore_barrier()

        def idx_fetch(slot, j):
            pltpu.async_copy(idx_hbm.at[wid, j], idxb.at[slot], igsem[slot])

        def idx_wait(slot, j):
            pltpu.make_async_copy(
                idx_hbm.at[wid, j], idxb.at[slot], igsem[slot]).wait()

        def gather_start(slot):
            pltpu.async_copy(
                table_hbm.at[idxb.at[slot, g_row]], rows[slot], gsem[slot])
            if side_gather:
                pltpu.async_copy(
                    side_hbm.at[idxb.at[slot, g_row]], srow[slot], ssem[slot])

        def gather_wait(slot):
            pltpu.make_async_copy(
                table_hbm.at[idxb.at[slot, g_row]], rows[slot], gsem[slot]).wait()
            if side_gather:
                pltpu.make_async_copy(
                    side_hbm.at[idxb.at[slot, g_row]], srow[slot], ssem[slot]).wait()

        def scatter(slot):
            sb = srow[slot] if side_gather else srow0
            pltpu.async_copy(rows[slot], acc_sh.at[idxb.at[slot, s_row]],
                             issem[slot], add=True)
            pltpu.async_copy(sb, accs_sh.at[idxb.at[slot, s_row]],
                             issem[slot], add=True)
            pltpu.make_async_copy(rows[slot], acc_sh.at[idxb.at[slot, s_row]],
                                  issem[slot]).wait()
            pltpu.make_async_copy(sb, accs_sh.at[idxb.at[slot, s_row]],
                                  issem[slot]).wait()

        idx_fetch(0, 0)
        idx_wait(0, 0)
        gather_start(0)
        idx_fetch(1, 1)
        idx_wait(1, 1)
        gather_start(1)
        idx_fetch(2, 2)

        def loop_body(t, carry):
            for b in range(3):
                c = 3 * t + b
                b2 = (b + 2) % 3
                gather_wait(b)
                scatter(b)
                @pl.when(c + 3 < chunks)
                def _():
                    idx_fetch(b, c + 3)
                idx_wait(b2, c + 2)
                gather_start(b2)
            return carry
        lax.fori_loop(0, chunks // 3, loop_body, 0)

        for b in range(2):
            gather_wait(b)
            scatter(b)

        plsc.subcore_barrier()
        pltpu.sync_copy(acc_sh.at[pl.ds(base_row, rows_per_tile)],
                        out_hbm.at[cid, pl.ds(base_row, rows_per_tile)])
        pltpu.sync_copy(accs_sh.at[pl.ds(base_row, rows_per_tile)],
                        outs_hbm.at[cid, pl.ds(base_row, rows_per_tile)])

    if side_gather:
        fn = body
    else:
        def fn(table_hbm, idx_hbm, out_hbm, outs_hbm,
               idxb, rows0, rows1, rows2, srow0, srow1, srow2, zbuf_v, zbufs_v,
               acc_sh, accs_sh,
               g0, g1, g2, s0, s1, s2, ig0, ig1, ig2, is0, is1, is2):
            return body(table_hbm, idx_hbm, table_hbm, out_hbm, outs_hbm,
                        idxb, rows0, rows1, rows2, srow0, srow1, srow2,
                        zbuf_v, zbufs_v, acc_sh, accs_sh,
                        g0, g1, g2, s0, s1, s2, ig0, ig1, ig2, is0, is1, is2)

    return functools.partial(
        pl.kernel,
        mesh=mesh,
        compiler_params=pltpu.CompilerParams(use_tc_tiling_on_sc=False),
        out_type=out_type,
        scratch_types=scratch,
    )(fn)


def _mid_body(yp_ref, deg_ref, o_ref):
    ysum = yp_ref[0] + yp_ref[1]
    deg = deg_ref[...]
    binv = jnp.where(deg > 0, 1.0 / deg, 0.0)
    o_ref[...] = ysum * binv


def _final_body(ap_ref, deg_ref, w_ref, b_ref, g_ref, be_ref, o_ref):
    asum = ap_ref[0] + ap_ref[1]
    deg = deg_ref[...]
    dinv = jnp.where(deg > 0, 1.0 / deg, 0.0)
    a = asum * dinv
    h = jnp.dot(a, w_ref[...], preferred_element_type=jnp.float32,
                precision=lax.Precision.HIGHEST)
    h = jnp.maximum(h + b_ref[...], 0.0)
    mu = jnp.mean(h, axis=1, keepdims=True)
    var = jnp.mean((h - mu) ** 2, axis=1, keepdims=True)
    o_ref[...] = (h - mu) * lax.rsqrt(var + LN_EPS) * g_ref[...] + be_ref[...]


def kernel(x, edge, edge_weight, W0, b0, W1, b1, W2, b2, W3, b3, gamma, beta):
    sc_pass1 = _make_sc_pass(side_gather=False, g_row=0, s_row=1)
    sc_pass2 = _make_sc_pass(side_gather=True, g_row=1, s_row=0)

    info = plsc.get_sparse_core_info()
    nw = info.num_cores * info.num_subcores
    chunks = N_INC // (nw * K_CHUNK)
    e4 = edge.reshape(2, nw, chunks, K_CHUNK)
    idx1 = jnp.stack([e4[0], e4[1]], axis=2)

    ypart, cpart = sc_pass1(x, idx1)
    deg_e = (cpart[0, :, 0] + cpart[1, :, 0]).reshape(N_HEDGES, 1)

    br = 2000
    hf = N_HEADS * F_OUT
    yscaled = pl.pallas_call(
        _mid_body,
        grid=(N_HEDGES // br,),
        in_specs=[
            pl.BlockSpec((2, br, F_IN), lambda i: (0, i, 0)),
            pl.BlockSpec((br, 1), lambda i: (i, 0)),
        ],
        out_specs=pl.BlockSpec((br, F_IN), lambda i: (i, 0)),
        out_shape=jax.ShapeDtypeStruct((N_HEDGES, F_IN), jnp.float32),
    )(ypart, deg_e)

    ew16 = jnp.broadcast_to(edge_weight.reshape(N_HEDGES, 1),
                            (N_HEDGES, D_SIDE))
    apart, wpart = sc_pass2(yscaled, idx1, ew16)
    deg_v = (wpart[0, :, 0] + wpart[1, :, 0]).reshape(N_NODES, 1)

    wcat = jnp.concatenate([W0, W1, W2, W3], axis=1)
    bcat = jnp.concatenate([b0, b1, b2, b3]).reshape(1, hf)
    out = pl.pallas_call(
        _final_body,
        grid=(N_NODES // br,),
        in_specs=[
            pl.BlockSpec((2, br, F_IN), lambda i: (0, i, 0)),
            pl.BlockSpec((br, 1), lambda i: (i, 0)),
            pl.BlockSpec((F_IN, hf), lambda i: (0, 0)),
            pl.BlockSpec((1, hf), lambda i: (0, 0)),
            pl.BlockSpec((1, hf), lambda i: (0, 0)),
            pl.BlockSpec((1, hf), lambda i: (0, 0)),
        ],
        out_specs=pl.BlockSpec((br, hf), lambda i: (i, 0)),
        out_shape=jax.ShapeDtypeStruct((N_NODES, hf), jnp.float32),
    )(apart, deg_v, wcat, bcat, gamma.reshape(1, hf), beta.reshape(1, hf))

    return out


# speedup vs baseline: 1.2880x; 1.1080x over previous
import functools

import jax
import jax.numpy as jnp
from jax import lax
from jax.experimental import pallas as pl
from jax.experimental.pallas import tpu as pltpu
from jax.experimental.pallas import tpu_sc as plsc

N_NODES = 10000
N_HEDGES = 10000
N_INC = 320000
F_IN = 128
F_OUT = 128
N_HEADS = 4
LN_EPS = 1e-5

D_SIDE = 16
K_CHUNK = 80


def _make_sc_pass(side_gather: bool, g_row: int, s_row: int):
    info = plsc.get_sparse_core_info()
    nc, ns, nl = info.num_cores, info.num_subcores, info.num_lanes
    nw = nc * ns
    epw = N_INC // nw
    chunks = epw // K_CHUNK
    assert N_INC == nw * K_CHUNK * chunks
    assert chunks % 3 == 2
    rows_per_tile = N_HEDGES // ns
    zrows = 25
    assert rows_per_tile % zrows == 0

    mesh = plsc.VectorSubcoreMesh(core_axis_name="c", subcore_axis_name="s")

    scratch = [
        pltpu.VMEM((3, 2, K_CHUNK), jnp.int32),
        pltpu.VMEM((K_CHUNK, F_IN), jnp.float32),
        pltpu.VMEM((K_CHUNK, F_IN), jnp.float32),
        pltpu.VMEM((K_CHUNK, F_IN), jnp.float32),
        pltpu.VMEM((K_CHUNK, D_SIDE), jnp.float32),
        pltpu.VMEM((K_CHUNK, D_SIDE), jnp.float32),
        pltpu.VMEM((K_CHUNK, D_SIDE), jnp.float32),
        pltpu.VMEM((zrows, F_IN), jnp.float32),
        pltpu.VMEM((5 * zrows, D_SIDE), jnp.float32),
        pltpu.VMEM_SHARED((N_HEDGES, F_IN), jnp.float32),
        pltpu.VMEM_SHARED((N_HEDGES, D_SIDE), jnp.float32),
    ] + [pltpu.SemaphoreType.DMA] * 12

    out_type = (
        jax.ShapeDtypeStruct((nc, N_HEDGES, F_IN), jnp.float32),
        jax.ShapeDtypeStruct((nc, N_HEDGES, D_SIDE), jnp.float32),
    )

    def body(table_hbm, idx_hbm, side_hbm, out_hbm, outs_hbm,
             idxb, rows0, rows1, rows2, srow0, srow1, srow2, zbuf_v, zbufs_v,
             acc_sh, accs_sh,
             g0, g1, g2, s0, s1, s2, ig0, ig1, ig2, is0, is1, is2):
        cid = lax.axis_index("c")
        sid = lax.axis_index("s")
        wid = cid * ns + sid
        rows = (rows0, rows1, rows2)
        srow = (srow0, srow1, srow2)
        gsem = (g0, g1, g2)
        ssem = (s0, s1, s2)
        igsem = (ig0, ig1, ig2)
        issem = (is0, is1, is2)

        def zrow(i, carry):
            def zcol(j, c2):
                zbuf_v[i, pl.ds(j * nl, nl)] = jnp.zeros((nl,), jnp.float32)
                return c2
            return lax.fori_loop(0, F_IN // nl, zcol, carry)
        lax.fori_loop(0, zrows, zrow, 0)

        def zrow_s(i, carry):
            zbufs_v[i, pl.ds(0, nl)] = jnp.zeros((nl,), jnp.float32)
            return carry
        lax.fori_loop(0, 5 * zrows, zrow_s, 0)
        base_row = sid * rows_per_tile
        for t in range(rows_per_tile // zrows):
            pltpu.sync_copy(zbuf_v, acc_sh.at[pl.ds(base_row + t * zrows, zrows)])
        for t in range(rows_per_tile // (5 * zrows)):
            pltpu.sync_copy(
                zbufs_v,
                accs_sh.at[pl.ds(base_row + t * 5 * zrows, 5 * zrows)])
        if not side_gather:
            def onerow(i, carry):
                srow0[i, pl.ds(0, nl)] = jnp.ones((nl,), jnp.float32)
                return carry
            lax.fori_loop(0, K_CHUNK, onerow, 0)
        plsc.subcore_barrier()

        ebase = wid * epw

        def idx_fetch(slot, j):
            off = ebase + j * K_CHUNK
            pltpu.async_copy(idx_hbm.at[0, pl.ds(off, K_CHUNK)],
                             idxb.at[slot, 0], igsem[slot])
            pltpu.async_copy(idx_hbm.at[1, pl.ds(off, K_CHUNK)],
                             idxb.at[slot, 1], igsem[slot])

        def idx_wait(slot, j):
            off = ebase + j * K_CHUNK
            pltpu.make_async_copy(idx_hbm.at[0, pl.ds(off, K_CHUNK)],
                                  idxb.at[slot, 0], igsem[slot]).wait()
            pltpu.make_async_copy(idx_hbm.at[1, pl.ds(off, K_CHUNK)],
                                  idxb.at[slot, 1], igsem[slot]).wait()

        def gather_start(slot):
            pltpu.async_copy(
                table_hbm.at[idxb.at[slot, g_row]], rows[slot], gsem[slot])
            if side_gather:
                pltpu.async_copy(
                    side_hbm.at[idxb.at[slot, g_row]], srow[slot], ssem[slot])

        def gather_wait(slot):
            pltpu.make_async_copy(
                table_hbm.at[idxb.at[slot, g_row]], rows[slot], gsem[slot]).wait()
            if side_gather:
                pltpu.make_async_copy(
                    side_hbm.at[idxb.at[slot, g_row]], srow[slot], ssem[slot]).wait()

        def scatter(slot):
            sb = srow[slot] if side_gather else srow0
            pltpu.async_copy(rows[slot], acc_sh.at[idxb.at[slot, s_row]],
                             issem[slot], add=True)
            pltpu.async_copy(sb, accs_sh.at[idxb.at[slot, s_row]],
                             issem[slot], add=True)
            pltpu.make_async_copy(rows[slot], acc_sh.at[idxb.at[slot, s_row]],
                                  issem[slot]).wait()
            pltpu.make_async_copy(sb, accs_sh.at[idxb.at[slot, s_row]],
                                  issem[slot]).wait()

        idx_fetch(0, 0)
        idx_wait(0, 0)
        gather_start(0)
        idx_fetch(1, 1)
        idx_wait(1, 1)
        gather_start(1)
        idx_fetch(2, 2)

        def loop_body(t, carry):
            for b in range(3):
                c = 3 * t + b
                b2 = (b + 2) % 3
                gather_wait(b)
                scatter(b)
                @pl.when(c + 3 < chunks)
                def _():
                    idx_fetch(b, c + 3)
                idx_wait(b2, c + 2)
                gather_start(b2)
            return carry
        lax.fori_loop(0, chunks // 3, loop_body, 0)

        for b in range(2):
            gather_wait(b)
            scatter(b)

        plsc.subcore_barrier()
        pltpu.sync_copy(acc_sh.at[pl.ds(base_row, rows_per_tile)],
                        out_hbm.at[cid, pl.ds(base_row, rows_per_tile)])
        pltpu.sync_copy(accs_sh.at[pl.ds(base_row, rows_per_tile)],
                        outs_hbm.at[cid, pl.ds(base_row, rows_per_tile)])

    if side_gather:
        fn = body
    else:
        def fn(table_hbm, idx_hbm, out_hbm, outs_hbm,
               idxb, rows0, rows1, rows2, srow0, srow1, srow2, zbuf_v, zbufs_v,
               acc_sh, accs_sh,
               g0, g1, g2, s0, s1, s2, ig0, ig1, ig2, is0, is1, is2):
            return body(table_hbm, idx_hbm, table_hbm, out_hbm, outs_hbm,
                        idxb, rows0, rows1, rows2, srow0, srow1, srow2,
                        zbuf_v, zbufs_v, acc_sh, accs_sh,
                        g0, g1, g2, s0, s1, s2, ig0, ig1, ig2, is0, is1, is2)

    return functools.partial(
        pl.kernel,
        mesh=mesh,
        compiler_params=pltpu.CompilerParams(use_tc_tiling_on_sc=False),
        out_type=out_type,
        scratch_types=scratch,
    )(fn)


def _mid_body(yp_ref, deg_ref, o_ref):
    ysum = yp_ref[0] + yp_ref[1]
    deg = deg_ref[...]
    binv = jnp.where(deg > 0, 1.0 / deg, 0.0)
    o_ref[...] = ysum * binv


def _final_body(ap_ref, deg_ref, w_ref, b_ref, g_ref, be_ref, o_ref):
    asum = ap_ref[0] + ap_ref[1]
    deg = deg_ref[...]
    dinv = jnp.where(deg > 0, 1.0 / deg, 0.0)
    a = asum * dinv
    h = jnp.dot(a, w_ref[...], preferred_element_type=jnp.float32,
                precision=lax.Precision.HIGHEST)
    h = jnp.maximum(h + b_ref[...], 0.0)
    mu = jnp.mean(h, axis=1, keepdims=True)
    var = jnp.mean((h - mu) ** 2, axis=1, keepdims=True)
    o_ref[...] = (h - mu) * lax.rsqrt(var + LN_EPS) * g_ref[...] + be_ref[...]


def kernel(x, edge, edge_weight, W0, b0, W1, b1, W2, b2, W3, b3, gamma, beta):
    sc_pass1 = _make_sc_pass(side_gather=False, g_row=0, s_row=1)
    sc_pass2 = _make_sc_pass(side_gather=True, g_row=1, s_row=0)

    ypart, cpart = sc_pass1(x, edge)
    deg_e = (cpart[0, :, 0] + cpart[1, :, 0]).reshape(N_HEDGES, 1)

    br = 2000
    hf = N_HEADS * F_OUT
    yscaled = pl.pallas_call(
        _mid_body,
        grid=(N_HEDGES // br,),
        in_specs=[
            pl.BlockSpec((2, br, F_IN), lambda i: (0, i, 0)),
            pl.BlockSpec((br, 1), lambda i: (i, 0)),
        ],
        out_specs=pl.BlockSpec((br, F_IN), lambda i: (i, 0)),
        out_shape=jax.ShapeDtypeStruct((N_HEDGES, F_IN), jnp.float32),
    )(ypart, deg_e)

    ew16 = jnp.broadcast_to(edge_weight.reshape(N_HEDGES, 1),
                            (N_HEDGES, D_SIDE))
    apart, wpart = sc_pass2(yscaled, edge, ew16)
    deg_v = (wpart[0, :, 0] + wpart[1, :, 0]).reshape(N_NODES, 1)

    wcat = jnp.concatenate([W0, W1, W2, W3], axis=1)
    bcat = jnp.concatenate([b0, b1, b2, b3]).reshape(1, hf)
    out = pl.pallas_call(
        _final_body,
        grid=(N_NODES // br,),
        in_specs=[
            pl.BlockSpec((2, br, F_IN), lambda i: (0, i, 0)),
            pl.BlockSpec((br, 1), lambda i: (i, 0)),
            pl.BlockSpec((F_IN, hf), lambda i: (0, 0)),
            pl.BlockSpec((1, hf), lambda i: (0, 0)),
            pl.BlockSpec((1, hf), lambda i: (0, 0)),
            pl.BlockSpec((1, hf), lambda i: (0, 0)),
        ],
        out_specs=pl.BlockSpec((br, hf), lambda i: (i, 0)),
        out_shape=jax.ShapeDtypeStruct((N_NODES, hf), jnp.float32),
    )(apart, deg_v, wcat, bcat, gamma.reshape(1, hf), beta.reshape(1, hf))

    return out
